# baseline (speedup 1.0000x reference)
import functools

import jax
import jax.numpy as jnp
from jax import lax
from jax.experimental import pallas as pl
from jax.experimental.pallas import tpu as pltpu
from jax.experimental.pallas import tpu_sc as plsc

NC = 2
NS = 16
NW = NC * NS
CH = 128


NB = 4


def _make_sc_segment_sum(NF, NP, CPW, D, with_count, stage_table, nb=NB):
  mesh = plsc.VectorSubcoreMesh(core_axis_name="c", subcore_axis_name="s")
  out_type = [jax.ShapeDtypeStruct((NC, NP, D), jnp.float32)]
  scratch = [
      pltpu.VMEM((CPW, CH), jnp.int32),
      pltpu.VMEM((CPW, CH), jnp.int32),
      [pltpu.VMEM((CH, D), jnp.float32) for _ in range(nb)],
      pltpu.VMEM_SHARED((NF if stage_table else 1, D), jnp.float32),
      pltpu.VMEM_SHARED((NP, D), jnp.float32),
      [pltpu.SemaphoreType.DMA for _ in range(nb)],
      [pltpu.SemaphoreType.DMA for _ in range(nb)],
  ]
  if with_count:
    out_type.append(jax.ShapeDtypeStruct((NC, NP, 16), jnp.float32))
    scratch.append(pltpu.VMEM((CH, 16), jnp.float32))
    scratch.append(pltpu.VMEM_SHARED((NP, 16), jnp.float32))
    scratch.append(pltpu.SemaphoreType.DMA)

  rows_per_tile = NP // NS
  zchunks = NP // CH // NS
  NG = CPW // nb

  rpf = NF // NS

  def body(feat, srcidx, dstidx, *rest):
    if with_count:
      (out_acc, out_cnt, src_slab, dst_slab, rows, feat_sh, acc_sh,
       gsem, ssem, ones_v, cnt_sh, osem) = rest
    else:
      (out_acc, src_slab, dst_slab, rows, feat_sh, acc_sh, gsem, ssem) = rest
    c = lax.axis_index("c")
    s = lax.axis_index("s")
    wid = c * NS + s
    zero16 = jnp.zeros((16,), jnp.float32)
    one16 = jnp.ones((16,), jnp.float32)

    def zrow(i, _):
      for d4 in range(D // 16):
        rows[0][i, pl.ds(d4 * 16, 16)] = zero16
      return 0
    lax.fori_loop(0, CH, zrow, 0)
    for j in range(zchunks):
      pltpu.sync_copy(rows[0], acc_sh.at[pl.ds((s * zchunks + j) * CH, CH)])
    if with_count:
      def zone(i, _):
        ones_v[i, pl.ds(0, 16)] = zero16
        return 0
      lax.fori_loop(0, CH, zone, 0)
      for j in range(zchunks):
        pltpu.sync_copy(ones_v, cnt_sh.at[pl.ds((s * zchunks + j) * CH, CH)])
      def frow(i, _):
        ones_v[i, pl.ds(0, 16)] = one16
        return 0
      lax.fori_loop(0, CH, frow, 0)

    pltpu.sync_copy(srcidx.at[wid], src_slab)
    pltpu.sync_copy(dstidx.at[wid], dst_slab)
    if stage_table:
      pltpu.sync_copy(feat.at[pl.ds(s * rpf, rpf)],
                      feat_sh.at[pl.ds(s * rpf, rpf)])
    plsc.subcore_barrier()
    gsrc = feat_sh if stage_table else feat

    def group(gi, _):
      g = gi * nb
      for b in range(nb):
        @pl.when(gi > 0)
        def _(b=b):
          pltpu.make_async_copy(rows[b], acc_sh.at[dst_slab.at[0]],
                                ssem[b]).wait()
        pltpu.async_copy(gsrc.at[src_slab.at[g + b]], rows[b], gsem[b])
      for b in range(nb):
        pltpu.make_async_copy(gsrc.at[src_slab.at[g + b]], rows[b],
                              gsem[b]).wait()
        pltpu.async_copy(rows[b], acc_sh.at[dst_slab.at[g + b]], ssem[b],
                         add=True)
        if with_count:
          pltpu.async_copy(ones_v, cnt_sh.at[dst_slab.at[g + b]], osem,
                           add=True)
      return 0

    lax.fori_loop(0, NG, group, 0)
    for b in range(nb):
      pltpu.make_async_copy(rows[b], acc_sh.at[dst_slab.at[0]], ssem[b]).wait()
    if with_count:
      def drain(i, _):
        pltpu.make_async_copy(ones_v, cnt_sh.at[dst_slab.at[0]], osem).wait()
        return 0
      lax.fori_loop(0, CPW, drain, 0)

    plsc.subcore_barrier()
    r0 = s * rows_per_tile
    pltpu.sync_copy(acc_sh.at[pl.ds(r0, rows_per_tile)],
                    out_acc.at[c, pl.ds(r0, rows_per_tile)])
    if with_count:
      pltpu.sync_copy(cnt_sh.at[pl.ds(r0, rows_per_tile)],
                      out_cnt.at[c, pl.ds(r0, rows_per_tile)])

  return pl.kernel(
      body, out_type=out_type, mesh=mesh, scratch_types=scratch,
      compiler_params=pltpu.CompilerParams(use_tc_tiling_on_sc=False))


def _pre_body(x_ref, wl_ref, wr_ref, p_ref, r_ref):
  xb = x_ref[...]
  p_ref[...] = jnp.dot(xb, wl_ref[...], preferred_element_type=jnp.float32)
  r_ref[...] = jnp.dot(xb, wr_ref[...], preferred_element_type=jnp.float32)


def _mid_body(acc_ref, cnt_ref, r_ref, b_ref, h_ref):
  a = acc_ref[0] + acc_ref[1]
  cnt = cnt_ref[0, :, 0:1] + cnt_ref[1, :, 0:1]
  mean = a / jnp.maximum(cnt, 1.0)
  h_ref[...] = jnp.maximum(mean + b_ref[...] + r_ref[...], 0.0)


def _post_body(acc_ref, cnt_ref, h_ref, wl_ref, wr_ref, b_ref, o_ref):
  a = acc_ref[0] + acc_ref[1]
  cnt = cnt_ref[0, :, 0:1] + cnt_ref[1, :, 0:1]
  mean = a / jnp.maximum(cnt, 1.0)
  hb = h_ref[...]
  z = (jnp.dot(mean, wl_ref[...], preferred_element_type=jnp.float32)
       + jnp.dot(hb, wr_ref[...], preferred_element_type=jnp.float32)
       + b_ref[...])
  m = jnp.max(z, axis=1, keepdims=True)
  lse = jnp.log(jnp.sum(jnp.exp(z - m), axis=1, keepdims=True))
  o_ref[...] = z - m - lse


@jax.jit
def kernel(x, edge_index, Wl1, bl1, Wr1, Wl2, bl2, Wr2):
  N, F = x.shape
  H = Wl1.shape[0]
  C = Wl2.shape[0]
  E = edge_index.shape[1]

  CPW = -(-(-(-E // (NW * CH))) // NB) * NB
  EP = NW * CPW * CH
  NP = -(-(N + 1) // (NS * CH)) * (NS * CH)

  pad_dst = N + jnp.arange(EP - E, dtype=jnp.int32) % (NP - N)
  pad_src = jnp.arange(EP - E, dtype=jnp.int32) % N
  src = jnp.concatenate([edge_index[0], pad_src]).reshape(NW, CPW, CH)
  dst = jnp.concatenate([edge_index[1], pad_dst]).reshape(NW, CPW, CH)

  RB = 1000
  grid = -(-N // RB)

  p1, r1 = pl.pallas_call(
      _pre_body,
      grid=(grid,),
      in_specs=[
          pl.BlockSpec((RB, F), lambda i: (i, 0)),
          pl.BlockSpec((F, H), lambda i: (0, 0)),
          pl.BlockSpec((F, H), lambda i: (0, 0)),
      ],
      out_specs=[
          pl.BlockSpec((RB, H), lambda i: (i, 0)),
          pl.BlockSpec((RB, H), lambda i: (i, 0)),
      ],
      out_shape=[
          jax.ShapeDtypeStruct((N, H), jnp.float32),
          jax.ShapeDtypeStruct((N, H), jnp.float32),
      ],
  )(x, Wl1.T, Wr1.T)

  acc1, cntacc = _make_sc_segment_sum(N, NP, CPW, H, True, True,
                                      nb=2)(p1, src, dst)

  h = pl.pallas_call(
      _mid_body,
      grid=(grid,),
      in_specs=[
          pl.BlockSpec((NC, RB, H), lambda i: (0, i, 0)),
          pl.BlockSpec((NC, RB, 16), lambda i: (0, i, 0)),
          pl.BlockSpec((RB, H), lambda i: (i, 0)),
          pl.BlockSpec((1, H), lambda i: (0, 0)),
      ],
      out_specs=pl.BlockSpec((RB, H), lambda i: (i, 0)),
      out_shape=jax.ShapeDtypeStruct((N, H), jnp.float32),
  )(acc1, cntacc, r1, bl1.reshape(1, H))

  (acc2,) = _make_sc_segment_sum(N, NP, CPW, H, False, True, nb=2)(h, src, dst)

  out = pl.pallas_call(
      _post_body,
      grid=(grid,),
      in_specs=[
          pl.BlockSpec((NC, RB, H), lambda i: (0, i, 0)),
          pl.BlockSpec((NC, RB, 16), lambda i: (0, i, 0)),
          pl.BlockSpec((RB, H), lambda i: (i, 0)),
          pl.BlockSpec((H, C), lambda i: (0, 0)),
          pl.BlockSpec((H, C), lambda i: (0, 0)),
          pl.BlockSpec((1, C), lambda i: (0, 0)),
      ],
      out_specs=pl.BlockSpec((RB, C), lambda i: (i, 0)),
      out_shape=jax.ShapeDtypeStruct((N, C), jnp.float32),
  )(acc2, cntacc, h, Wl2.T, Wr2.T, bl2.reshape(1, C))

  return out


# speedup vs baseline: 1.1983x; 1.1983x over previous
import functools

import jax
import jax.numpy as jnp
from jax import lax
from jax.experimental import pallas as pl
from jax.experimental.pallas import tpu as pltpu
from jax.experimental.pallas import tpu_sc as plsc

NC = 2
NS = 16
NW = NC * NS
CH = 128


NB = 4


def _make_sc_segment_sum(NF, NP, CPW, D, with_count, stage_table, nb=NB):
  mesh = plsc.VectorSubcoreMesh(core_axis_name="c", subcore_axis_name="s")
  out_type = [jax.ShapeDtypeStruct((NC, NP, D), jnp.float32)]
  scratch = [
      pltpu.VMEM((CPW, CH), jnp.int32),
      pltpu.VMEM((CPW, CH), jnp.int32),
      [pltpu.VMEM((CH, D), jnp.float32) for _ in range(nb)],
      pltpu.VMEM_SHARED((NF if stage_table else 1, D), jnp.float32),
      pltpu.VMEM_SHARED((NP, D), jnp.float32),
      [pltpu.SemaphoreType.DMA for _ in range(nb)],
      [pltpu.SemaphoreType.DMA for _ in range(nb)],
  ]
  if with_count:
    out_type.append(jax.ShapeDtypeStruct((NC, NP, 16), jnp.float32))
    scratch.append(pltpu.VMEM((CH, 16), jnp.float32))
    scratch.append(pltpu.VMEM_SHARED((NP, 16), jnp.float32))
    scratch.append(pltpu.SemaphoreType.DMA)

  rows_per_tile = NP // NS
  zchunks = NP // CH // NS
  NG = CPW // nb

  rpf = NF // NS

  def body(feat, srcidx, dstidx, *rest):
    if with_count:
      (out_acc, out_cnt, src_slab, dst_slab, rows, feat_sh, acc_sh,
       gsem, ssem, ones_v, cnt_sh, osem) = rest
    else:
      (out_acc, src_slab, dst_slab, rows, feat_sh, acc_sh, gsem, ssem) = rest
    c = lax.axis_index("c")
    s = lax.axis_index("s")
    wid = c * NS + s
    zero16 = jnp.zeros((16,), jnp.float32)
    one16 = jnp.ones((16,), jnp.float32)

    def zrow(i, _):
      for d4 in range(D // 16):
        rows[0][i, pl.ds(d4 * 16, 16)] = zero16
      return 0
    lax.fori_loop(0, CH, zrow, 0)
    for j in range(zchunks):
      pltpu.sync_copy(rows[0], acc_sh.at[pl.ds((s * zchunks + j) * CH, CH)])
    if with_count:
      def zone(i, _):
        ones_v[i, pl.ds(0, 16)] = zero16
        return 0
      lax.fori_loop(0, CH, zone, 0)
      for j in range(zchunks):
        pltpu.sync_copy(ones_v, cnt_sh.at[pl.ds((s * zchunks + j) * CH, CH)])
      def frow(i, _):
        ones_v[i, pl.ds(0, 16)] = one16
        return 0
      lax.fori_loop(0, CH, frow, 0)

    pltpu.sync_copy(srcidx.at[wid], src_slab)
    pltpu.sync_copy(dstidx.at[wid], dst_slab)
    if stage_table:
      pltpu.sync_copy(feat.at[pl.ds(s * rpf, rpf)],
                      feat_sh.at[pl.ds(s * rpf, rpf)])
    plsc.subcore_barrier()
    gsrc = feat_sh if stage_table else feat

    def group(gi, _):
      g = gi * nb
      for b in range(nb):
        @pl.when(gi > 0)
        def _(b=b):
          pltpu.make_async_copy(rows[b], acc_sh.at[dst_slab.at[0]],
                                ssem[b]).wait()
        pltpu.async_copy(gsrc.at[src_slab.at[g + b]], rows[b], gsem[b])
      for b in range(nb):
        pltpu.make_async_copy(gsrc.at[src_slab.at[g + b]], rows[b],
                              gsem[b]).wait()
        pltpu.async_copy(rows[b], acc_sh.at[dst_slab.at[g + b]], ssem[b],
                         add=True)
        if with_count:
          pltpu.async_copy(ones_v, cnt_sh.at[dst_slab.at[g + b]], osem,
                           add=True)
      return 0

    lax.fori_loop(0, NG, group, 0)
    for b in range(nb):
      pltpu.make_async_copy(rows[b], acc_sh.at[dst_slab.at[0]], ssem[b]).wait()
    if with_count:
      def drain(i, _):
        pltpu.make_async_copy(ones_v, cnt_sh.at[dst_slab.at[0]], osem).wait()
        return 0
      lax.fori_loop(0, CPW, drain, 0)

    plsc.subcore_barrier()
    r0 = s * rows_per_tile
    pltpu.sync_copy(acc_sh.at[pl.ds(r0, rows_per_tile)],
                    out_acc.at[c, pl.ds(r0, rows_per_tile)])
    if with_count:
      pltpu.sync_copy(cnt_sh.at[pl.ds(r0, rows_per_tile)],
                      out_cnt.at[c, pl.ds(r0, rows_per_tile)])

  return pl.kernel(
      body, out_type=out_type, mesh=mesh, scratch_types=scratch,
      compiler_params=pltpu.CompilerParams(use_tc_tiling_on_sc=False))


def _pre_body(x_ref, wl_ref, wr_ref, p_ref, r_ref):
  xb = x_ref[...]
  p_ref[...] = jnp.dot(xb, wl_ref[...], preferred_element_type=jnp.float32)
  r_ref[...] = jnp.dot(xb, wr_ref[...], preferred_element_type=jnp.float32)


def _mid_body(acc_ref, cnt_ref, r_ref, b_ref, h_ref):
  a = acc_ref[0] + acc_ref[1]
  cnt = cnt_ref[0, :, 0:1] + cnt_ref[1, :, 0:1]
  mean = a / jnp.maximum(cnt, 1.0)
  h_ref[...] = jnp.maximum(mean + b_ref[...] + r_ref[...], 0.0)


def _post_body(acc_ref, cnt_ref, h_ref, wl_ref, wr_ref, b_ref, o_ref):
  a = acc_ref[0] + acc_ref[1]
  cnt = cnt_ref[0, :, 0:1] + cnt_ref[1, :, 0:1]
  mean = a / jnp.maximum(cnt, 1.0)
  hb = h_ref[...]
  z = (jnp.dot(mean, wl_ref[...], preferred_element_type=jnp.float32)
       + jnp.dot(hb, wr_ref[...], preferred_element_type=jnp.float32)
       + b_ref[...])
  m = jnp.max(z, axis=1, keepdims=True)
  lse = jnp.log(jnp.sum(jnp.exp(z - m), axis=1, keepdims=True))
  o_ref[...] = z - m - lse


@jax.jit
def kernel(x, edge_index, Wl1, bl1, Wr1, Wl2, bl2, Wr2):
  N, F = x.shape
  H = Wl1.shape[0]
  C = Wl2.shape[0]
  E = edge_index.shape[1]

  CPW = -(-(-(-E // (NW * CH))) // NB) * NB
  EP = NW * CPW * CH
  NP = -(-(N + 1) // (NS * CH)) * (NS * CH)

  pad_dst = N + jnp.arange(EP - E, dtype=jnp.int32) % (NP - N)
  pad_src = jnp.arange(EP - E, dtype=jnp.int32) % N
  src = jnp.concatenate([edge_index[0], pad_src]).reshape(NW, CPW, CH)
  dst = jnp.concatenate([edge_index[1], pad_dst]).reshape(NW, CPW, CH)

  RB = 1000
  grid = -(-N // RB)

  p1, r1 = pl.pallas_call(
      _pre_body,
      grid=(grid,),
      in_specs=[
          pl.BlockSpec((RB, F), lambda i: (i, 0)),
          pl.BlockSpec((F, H), lambda i: (0, 0)),
          pl.BlockSpec((F, H), lambda i: (0, 0)),
      ],
      out_specs=[
          pl.BlockSpec((RB, H), lambda i: (i, 0)),
          pl.BlockSpec((RB, H), lambda i: (i, 0)),
      ],
      out_shape=[
          jax.ShapeDtypeStruct((N, H), jnp.float32),
          jax.ShapeDtypeStruct((N, H), jnp.float32),
      ],
  )(x, Wl1.T, Wr1.T)

  acc1, cntacc = _make_sc_segment_sum(N, NP, CPW, H, True, False,
                                      nb=5)(p1, src, dst)

  h = pl.pallas_call(
      _mid_body,
      grid=(grid,),
      in_specs=[
          pl.BlockSpec((NC, RB, H), lambda i: (0, i, 0)),
          pl.BlockSpec((NC, RB, 16), lambda i: (0, i, 0)),
          pl.BlockSpec((RB, H), lambda i: (i, 0)),
          pl.BlockSpec((1, H), lambda i: (0, 0)),
      ],
      out_specs=pl.BlockSpec((RB, H), lambda i: (i, 0)),
      out_shape=jax.ShapeDtypeStruct((N, H), jnp.float32),
  )(acc1, cntacc, r1, bl1.reshape(1, H))

  (acc2,) = _make_sc_segment_sum(N, NP, CPW, H, False, True, nb=2)(h, src, dst)

  out = pl.pallas_call(
      _post_body,
      grid=(grid,),
      in_specs=[
          pl.BlockSpec((NC, RB, H), lambda i: (0, i, 0)),
          pl.BlockSpec((NC, RB, 16), lambda i: (0, i, 0)),
          pl.BlockSpec((RB, H), lambda i: (i, 0)),
          pl.BlockSpec((H, C), lambda i: (0, 0)),
          pl.BlockSpec((H, C), lambda i: (0, 0)),
          pl.BlockSpec((1, C), lambda i: (0, 0)),
      ],
      out_specs=pl.BlockSpec((RB, C), lambda i: (i, 0)),
      out_shape=jax.ShapeDtypeStruct((N, C), jnp.float32),
  )(acc2, cntacc, h, Wl2.T, Wr2.T, bl2.reshape(1, C))

  return out
